# baseline (speedup 1.0000x reference)
import functools

import jax
import jax.numpy as jnp
from jax import lax
from jax.experimental import pallas as pl
from jax.experimental.pallas import tpu as pltpu
from jax.experimental.pallas import tpu_sc as plsc

D_EMBED = 512
BATCH = 16384
NUM_CORES = 2
NUM_SUBCORES = 16
ROWS_C0 = 488
ROWS_C1 = 536
CHUNKS_C0 = (112, 112, 112, 112, 40)
CHUNKS_C1 = (112, 112, 112, 112, 88)
CHUNK_MAX = 112
NBUF = 2

_mesh = plsc.VectorSubcoreMesh(core_axis_name="c", subcore_axis_name="s")


@functools.partial(
    pl.kernel,
    mesh=_mesh,
    out_type=jax.ShapeDtypeStruct((BATCH, D_EMBED), jnp.float32),
    scratch_types=[
        pltpu.VMEM((max(ROWS_C0, ROWS_C1),), jnp.int32),
        pltpu.VMEM((NBUF, CHUNK_MAX, D_EMBED), jnp.float32),
        pltpu.SemaphoreType.DMA,
        pltpu.SemaphoreType.DMA,
        pltpu.SemaphoreType.DMA,
        pltpu.SemaphoreType.DMA,
    ],
)
def _sc_gather(table_hbm, idx_hbm, out_hbm, idx_v, rows_v,
               g0, g1, w0, w1):
    cid = lax.axis_index("c")
    sid = lax.axis_index("s")
    gsems = (g0, g1)
    wsems = (w0, w1)

    def pipeline(base, n_rows, sizes):
        starts = [sum(sizes[:k]) for k in range(len(sizes))]
        pltpu.sync_copy(idx_hbm.at[pl.ds(base, n_rows)], idx_v.at[pl.ds(0, n_rows)])

        def gather(i):
            b = i % NBUF
            return pltpu.async_copy(
                table_hbm.at[idx_v.at[pl.ds(starts[i], sizes[i])]],
                rows_v.at[b, pl.ds(0, sizes[i])],
                gsems[b],
            )

        def write(i):
            b = i % NBUF
            return pltpu.async_copy(
                rows_v.at[b, pl.ds(0, sizes[i])],
                out_hbm.at[pl.ds(base + starts[i], sizes[i])],
                wsems[b],
            )

        n = len(sizes)
        gh = {}
        wh = {}
        gh[0] = gather(0)
        for i in range(n):
            if i + 1 < n:
                if i - 1 >= 0:
                    wh[i - 1].wait()
                gh[i + 1] = gather(i + 1)
            gh[i].wait()
            wh[i] = write(i)
        wh[n - 2].wait()
        wh[n - 1].wait()

    @pl.when(cid == 0)
    def _():
        pipeline(sid * ROWS_C0, ROWS_C0, CHUNKS_C0)

    @pl.when(cid == 1)
    def _():
        pipeline(NUM_SUBCORES * ROWS_C0 + sid * ROWS_C1, ROWS_C1, CHUNKS_C1)


def kernel(t, embedding_table):
    return _sc_gather(embedding_table, t.astype(jnp.int32))


# speedup vs baseline: 1.0245x; 1.0245x over previous
import functools

import jax
import jax.numpy as jnp
from jax import lax
from jax.experimental import pallas as pl
from jax.experimental.pallas import tpu as pltpu
from jax.experimental.pallas import tpu_sc as plsc

D_EMBED = 512
BATCH = 16384
NUM_CORES = 2
NUM_SUBCORES = 16
NUM_WORKERS = NUM_CORES * NUM_SUBCORES
B_PER_W = BATCH // NUM_WORKERS
CHUNK_SIZES = (112, 112, 112, 112, 64)
CHUNK_STARTS = (0, 112, 224, 336, 448)
CHUNK_MAX = 112
NBUF = 2
NCHUNK = len(CHUNK_SIZES)

_mesh = plsc.VectorSubcoreMesh(core_axis_name="c", subcore_axis_name="s")


@functools.partial(
    pl.kernel,
    mesh=_mesh,
    out_type=jax.ShapeDtypeStruct((BATCH, D_EMBED), jnp.float32),
    scratch_types=[
        pltpu.VMEM((B_PER_W,), jnp.int32),
        pltpu.VMEM((NBUF, CHUNK_MAX, D_EMBED), jnp.float32),
        pltpu.SemaphoreType.DMA,
        pltpu.SemaphoreType.DMA,
        pltpu.SemaphoreType.DMA,
        pltpu.SemaphoreType.DMA,
    ],
)
def _sc_gather(table_hbm, idx_hbm, out_hbm, idx_v, rows_v,
               g0, g1, w0, w1):
    wid = lax.axis_index("s") * NUM_CORES + lax.axis_index("c")
    base = wid * B_PER_W
    gsems = (g0, g1)
    wsems = (w0, w1)

    pltpu.sync_copy(idx_hbm.at[pl.ds(base, B_PER_W)], idx_v)

    def gather(i):
        b = i % NBUF
        return pltpu.async_copy(
            table_hbm.at[idx_v.at[pl.ds(CHUNK_STARTS[i], CHUNK_SIZES[i])]],
            rows_v.at[b, pl.ds(0, CHUNK_SIZES[i])],
            gsems[b],
        )

    def write(i):
        b = i % NBUF
        return pltpu.async_copy(
            rows_v.at[b, pl.ds(0, CHUNK_SIZES[i])],
            out_hbm.at[pl.ds(base + CHUNK_STARTS[i], CHUNK_SIZES[i])],
            wsems[b],
        )

    gh = {}
    wh = {}
    gh[0] = gather(0)
    for i in range(NCHUNK):
        if i + 1 < NCHUNK:
            if i - 1 >= 0:
                wh[i - 1].wait()
            gh[i + 1] = gather(i + 1)
        gh[i].wait()
        wh[i] = write(i)
    wh[NCHUNK - 2].wait()
    wh[NCHUNK - 1].wait()


def kernel(t, embedding_table):
    return _sc_gather(embedding_table, t.astype(jnp.int32))
